# baseline (speedup 1.0000x reference)
import functools

import jax
import jax.numpy as jnp
from jax import lax
from jax.experimental import pallas as pl
from jax.experimental.pallas import tpu as pltpu
from jax.experimental.pallas import tpu_sc as plsc

_NUM_CORES = 2
_NUM_SUBCORES = 16
_NUM_WORKERS = _NUM_CORES * _NUM_SUBCORES


@functools.lru_cache(maxsize=None)
def _make_gather(total, vocab, dim, chunk):
    b_per_w = total // _NUM_WORKERS
    n_chunks = b_per_w // chunk
    mesh = plsc.VectorSubcoreMesh(core_axis_name="c", subcore_axis_name="s")

    @functools.partial(
        pl.kernel,
        mesh=mesh,
        out_type=jax.ShapeDtypeStruct((total, dim), jnp.float32),
        scratch_types=[
            pltpu.VMEM((chunk,), jnp.int32),
            pltpu.VMEM((chunk, dim), jnp.float32),
            pltpu.SemaphoreType.DMA,
        ],
        compiler_params=pltpu.CompilerParams(use_tc_tiling_on_sc=False),
    )
    def gather_kernel(idx_hbm, table_hbm, out_hbm, idx_v, rows_v, sem):
        wid = lax.axis_index("s") * _NUM_CORES + lax.axis_index("c")
        base = wid * b_per_w

        def body(ci, carry):
            off = base + ci * chunk
            pltpu.sync_copy(idx_hbm.at[pl.ds(off, chunk)], idx_v)
            pltpu.async_copy(table_hbm.at[idx_v], rows_v, sem).wait()
            pltpu.sync_copy(rows_v, out_hbm.at[pl.ds(off, chunk)])
            return carry

        lax.fori_loop(0, n_chunks, body, 0)

    return gather_kernel


def kernel(input_ids, table):
    batch, hist = input_ids.shape
    vocab, dim = table.shape
    total = batch * hist
    ids = input_ids.reshape(total).astype(jnp.int32)
    out = _make_gather(total, vocab, dim, 512)(ids, table)
    return out.reshape(batch, hist, dim)


# speedup vs baseline: 1.0759x; 1.0759x over previous
import functools

import jax
import jax.numpy as jnp
from jax import lax
from jax.experimental import pallas as pl
from jax.experimental.pallas import tpu as pltpu
from jax.experimental.pallas import tpu_sc as plsc

_NUM_CORES = 2
_NUM_SUBCORES = 16
_NUM_WORKERS = _NUM_CORES * _NUM_SUBCORES
_NBUF = 2


@functools.lru_cache(maxsize=None)
def _make_gather(total, vocab, dim, chunk):
    b_per_w = total // _NUM_WORKERS
    n_chunks = b_per_w // chunk
    n_groups = n_chunks // _NBUF
    mesh = plsc.VectorSubcoreMesh(core_axis_name="c", subcore_axis_name="s")

    scratch = (
        [pltpu.VMEM((chunk,), jnp.int32) for _ in range(_NBUF)]
        + [pltpu.VMEM((chunk, dim), jnp.float32) for _ in range(_NBUF)]
        + [pltpu.SemaphoreType.DMA for _ in range(3 * _NBUF)]
    )

    @functools.partial(
        pl.kernel,
        mesh=mesh,
        out_type=jax.ShapeDtypeStruct((total, dim), jnp.float32),
        scratch_types=scratch,
        compiler_params=pltpu.CompilerParams(use_tc_tiling_on_sc=False),
    )
    def gather_kernel(idx_hbm, table_hbm, out_hbm, *bufs):
        idx_bufs = bufs[0:_NBUF]
        row_bufs = bufs[_NBUF : 2 * _NBUF]
        idx_sems = bufs[2 * _NBUF : 3 * _NBUF]
        g_sems = bufs[3 * _NBUF : 4 * _NBUF]
        out_sems = bufs[4 * _NBUF : 5 * _NBUF]

        wid = lax.axis_index("s") * _NUM_CORES + lax.axis_index("c")
        base = wid * b_per_w

        def start_idx(ci, b):
            pltpu.async_copy(
                idx_hbm.at[pl.ds(base + ci * chunk, chunk)], idx_bufs[b], idx_sems[b]
            )

        def wait_idx(b):
            pltpu.make_async_copy(
                idx_hbm.at[pl.ds(0, chunk)], idx_bufs[b], idx_sems[b]
            ).wait()

        def start_gather(b):
            pltpu.async_copy(table_hbm.at[idx_bufs[b]], row_bufs[b], g_sems[b])

        def wait_gather(b):
            pltpu.make_async_copy(
                table_hbm.at[pl.ds(0, chunk)], row_bufs[b], g_sems[b]
            ).wait()

        def start_out(ci, b):
            pltpu.async_copy(
                row_bufs[b], out_hbm.at[pl.ds(base + ci * chunk, chunk)], out_sems[b]
            )

        def wait_out(b):
            pltpu.make_async_copy(
                row_bufs[b], out_hbm.at[pl.ds(0, chunk)], out_sems[b]
            ).wait()

        for b in range(_NBUF):
            start_idx(b, b)

        def outer(g, carry):
            for b in range(_NBUF):
                ci = g * _NBUF + b
                @pl.when(g > 0)
                def _():
                    wait_out(b)

                wait_idx(b)
                start_gather(b)
                wait_gather(b)
                start_out(ci, b)

                @pl.when(g < n_groups - 1)
                def _():
                    start_idx(ci + _NBUF, b)

            return carry

        lax.fori_loop(0, n_groups, outer, 0, unroll=False)

        for b in range(_NBUF):
            wait_out(b)

    return gather_kernel


def kernel(input_ids, table):
    batch, hist = input_ids.shape
    vocab, dim = table.shape
    total = batch * hist
    ids = input_ids.reshape(total).astype(jnp.int32)
    out = _make_gather(total, vocab, dim, 800)(ids, table)
    return out.reshape(batch, hist, dim)
